# baseline (speedup 1.0000x reference)
import functools

import jax
import jax.numpy as jnp
from jax import lax
from jax.experimental import pallas as pl
from jax.experimental.pallas import tpu as pltpu
from jax.experimental.pallas import tpu_sc as plsc

N = 10000
D = 128
L = 5
G = 256
E = 320000

BLK = 1000
NB = N // BLK

NW = 16
C = 128
CPT = 160
E_PAD = NW * C * CPT
N_ACC = 10240
RPT = N_ACC // 16


def _k0_body(x_ref, w_ref, b_ref, h_ref):
    h_ref[...] = jnp.maximum(
        jnp.dot(x_ref[...], w_ref[...], preferred_element_type=jnp.float32)
        + b_ref[...], 0.0)


def _stats_update(i, s_ref, y):
    @pl.when(i == 0)
    def _():
        c0 = jnp.sum(y, axis=0, keepdims=True) * (1.0 / BLK)
        s_ref[...] = jnp.concatenate(
            [jnp.zeros((2, D), jnp.float32), c0,
             jnp.zeros((5, D), jnp.float32)], axis=0)

    c = s_ref[2:3, :]
    yc = y - c
    s_ref[...] += jnp.concatenate(
        [jnp.sum(yc, axis=0, keepdims=True),
         jnp.sum(yc * yc, axis=0, keepdims=True),
         jnp.zeros((6, D), jnp.float32)], axis=0)


def _ka_body(scale_ref, h_ref, a0_ref, a1_ref, w_ref, b_ref, y_ref, s_ref):
    i = pl.program_id(0)
    z = scale_ref[0, 0] * h_ref[...] + a0_ref[0] + a1_ref[0]
    y = jnp.dot(z, w_ref[...], preferred_element_type=jnp.float32) + b_ref[...]
    y_ref[...] = y
    _stats_update(i, s_ref, y)


def _bn_relu(y, s_ref, g_ref, be_ref):
    s = s_ref[...]
    dm = s[0:1] * (1.0 / N)
    m = s[2:3] + dm
    v = s[1:2] * (1.0 / N) - dm * dm
    inv = 1.0 / jnp.sqrt(v + 1e-5)
    return jnp.maximum(g_ref[...] * (y - m) * inv + be_ref[...], 0.0)


def _kb_body(y1_ref, s1_ref, g_ref, be_ref, w_ref, b_ref, y2_ref, s2_ref):
    i = pl.program_id(0)
    t = _bn_relu(y1_ref[...], s1_ref, g_ref, be_ref)
    y2 = jnp.dot(t, w_ref[...], preferred_element_type=jnp.float32) + b_ref[...]
    y2_ref[...] = y2
    _stats_update(i, s2_ref, y2)


def _kc_body(y2_ref, s2_ref, g_ref, be_ref, h_ref):
    h_ref[...] = _bn_relu(y2_ref[...], s2_ref, g_ref, be_ref)


def _kpool_body(h_ref, b3_ref, p_ref, c_ref):
    i = pl.program_id(0)
    bv = b3_ref[0]
    iota = lax.broadcasted_iota(jnp.int32, (G, BLK), 0)
    oh = (iota == bv).astype(jnp.float32)

    @pl.when(i == 0)
    def _():
        p_ref[...] = jnp.zeros_like(p_ref)
        c_ref[...] = jnp.zeros_like(c_ref)

    p_ref[...] += jnp.dot(oh, h_ref[...], preferred_element_type=jnp.float32)
    c_ref[...] += jnp.concatenate(
        [jnp.sum(oh, axis=1, keepdims=True), jnp.zeros((G, 7), jnp.float32)],
        axis=1)


def _khead_body(p_ref, c_ref, w1_ref, b1_ref, w2_ref, b2_ref, o_ref):
    cnt = jnp.maximum(c_ref[...][:, 0:1], 1.0)
    mean = p_ref[...] / cnt
    t = jnp.maximum(
        jnp.dot(mean, w1_ref[...], preferred_element_type=jnp.float32)
        + b1_ref[...], 0.0)
    o_ref[...] = (
        jnp.dot(t, w2_ref[...], preferred_element_type=jnp.float32)
        + b2_ref[...])


def _full(shape):
    return pl.BlockSpec(shape, lambda i: (0,) * len(shape))


_ROWBLK = pl.BlockSpec((BLK, D), lambda i: (i, 0))


def _k0_call(x, w, b):
    return pl.pallas_call(
        _k0_body, grid=(NB,),
        in_specs=[_ROWBLK, _full((D, D)), _full((1, D))],
        out_specs=_ROWBLK,
        out_shape=jax.ShapeDtypeStruct((N, D), jnp.float32))(x, w, b)


def _ka_call(scale, h, agg, w, b):
    return pl.pallas_call(
        _ka_body, grid=(NB,),
        in_specs=[
            _full((1, 1)),
            _ROWBLK,
            pl.BlockSpec((1, BLK, D), lambda i: (0, i, 0)),
            pl.BlockSpec((1, BLK, D), lambda i: (1, i, 0)),
            _full((D, D)),
            _full((1, D)),
        ],
        out_specs=[_ROWBLK, pl.BlockSpec((8, D), lambda i: (0, 0))],
        out_shape=[
            jax.ShapeDtypeStruct((N, D), jnp.float32),
            jax.ShapeDtypeStruct((8, D), jnp.float32),
        ])(scale, h, agg, agg, w, b)


def _kb_call(y1, s1, g, be, w, b):
    return pl.pallas_call(
        _kb_body, grid=(NB,),
        in_specs=[
            _ROWBLK, _full((8, D)), _full((1, D)), _full((1, D)),
            _full((D, D)), _full((1, D)),
        ],
        out_specs=[_ROWBLK, pl.BlockSpec((8, D), lambda i: (0, 0))],
        out_shape=[
            jax.ShapeDtypeStruct((N, D), jnp.float32),
            jax.ShapeDtypeStruct((8, D), jnp.float32),
        ])(y1, s1, g, be, w, b)


def _kc_call(y2, s2, g, be):
    return pl.pallas_call(
        _kc_body, grid=(NB,),
        in_specs=[_ROWBLK, _full((8, D)), _full((1, D)), _full((1, D))],
        out_specs=_ROWBLK,
        out_shape=jax.ShapeDtypeStruct((N, D), jnp.float32))(y2, s2, g, be)


def _kpool_call(h, b3):
    return pl.pallas_call(
        _kpool_body, grid=(NB,),
        in_specs=[_ROWBLK, pl.BlockSpec((1, 1, BLK), lambda i: (i, 0, 0))],
        out_specs=[
            pl.BlockSpec((G, D), lambda i: (0, 0)),
            pl.BlockSpec((G, 8), lambda i: (0, 0)),
        ],
        out_shape=[
            jax.ShapeDtypeStruct((G, D), jnp.float32),
            jax.ShapeDtypeStruct((G, 8), jnp.float32),
        ])(h, b3)


def _khead_call(p, c, w1, b1, w2, b2):
    return pl.pallas_call(
        _khead_body,
        out_shape=jax.ShapeDtypeStruct((G, 1), jnp.float32))(
            p, c, w1, b1, w2, b2)


@functools.cache
def _get_sc_scatter():
    mesh = plsc.VectorSubcoreMesh(core_axis_name="c", subcore_axis_name="s")

    @functools.partial(
        pl.kernel, mesh=mesh,
        out_type=jax.ShapeDtypeStruct((2, N_ACC, D), jnp.float32),
        scratch_types=[
            pltpu.VMEM((C,), jnp.int32),
            pltpu.VMEM((C,), jnp.int32),
            pltpu.VMEM((C,), jnp.int32),
            pltpu.VMEM((C,), jnp.int32),
            pltpu.VMEM((C, D), jnp.float32),
            pltpu.VMEM((C, D), jnp.float32),
            pltpu.VMEM((16, D), jnp.float32),
            pltpu.VMEM_SHARED((N_ACC, D), jnp.float32),
            pltpu.SemaphoreType.DMA,
            pltpu.SemaphoreType.DMA,
            pltpu.SemaphoreType.DMA,
            pltpu.SemaphoreType.DMA,
            pltpu.SemaphoreType.DMA,
            pltpu.SemaphoreType.DMA,
        ])
    def sc_scatter(h_hbm, s3_hbm, d3_hbm, out_hbm,
                   sa, da, sb, db, rows_a, rows_b, zrows, acc,
                   ssem_a, dsem_a, ssem_b, dsem_b, gsem_a, gsem_b):
        cid = lax.axis_index("c")
        sid = lax.axis_index("s")
        wid = sid

        zero16 = jnp.zeros((16,), jnp.float32)
        for r in range(16):
            for c8 in range(8):
                zrows[r, pl.ds(c8 * 16, 16)] = zero16
        base = sid * RPT
        for j in range(RPT // 16):
            pltpu.sync_copy(zrows, acc.at[pl.ds(base + j * 16, 16)])
        plsc.subcore_barrier()

        def fetch(ref3, i, buf, sem):
            pltpu.async_copy(ref3.at[wid, i], buf, sem)

        def wait(ref3, i, buf, sem):
            pltpu.make_async_copy(ref3.at[wid, i], buf, sem).wait()

        @pl.when(cid == 0)
        def _edge_work():
            fetch(s3_hbm, 0, sa, ssem_a)
            fetch(d3_hbm, 0, da, dsem_a)
            fetch(s3_hbm, 1, sb, ssem_b)
            fetch(d3_hbm, 1, db, dsem_b)
            wait(s3_hbm, 0, sa, ssem_a)
            pltpu.async_copy(h_hbm.at[sa], rows_a, gsem_a)
            wait(s3_hbm, 1, sb, ssem_b)
            pltpu.async_copy(h_hbm.at[sb], rows_b, gsem_b)

            def slot_step(i, s_buf, d_buf, rows, ssem, dsem, gsem):
                pltpu.make_async_copy(h_hbm.at[s_buf], rows, gsem).wait()
                fetch(s3_hbm, i + 2, s_buf, ssem)
                wait(d3_hbm, i, d_buf, dsem)
                pltpu.sync_copy(rows, acc.at[d_buf], add=True)
                fetch(d3_hbm, i + 2, d_buf, dsem)
                wait(s3_hbm, i + 2, s_buf, ssem)
                pltpu.async_copy(h_hbm.at[s_buf], rows, gsem)

            def body(j, carry):
                i0 = 2 * j
                slot_step(i0, sa, da, rows_a, ssem_a, dsem_a, gsem_a)
                slot_step(i0 + 1, sb, db, rows_b, ssem_b, dsem_b, gsem_b)
                return carry

            lax.fori_loop(0, CPT // 2 - 1, body, 0)
            pltpu.make_async_copy(h_hbm.at[sa], rows_a, gsem_a).wait()
            wait(d3_hbm, CPT - 2, da, dsem_a)
            pltpu.sync_copy(rows_a, acc.at[da], add=True)
            pltpu.make_async_copy(h_hbm.at[sb], rows_b, gsem_b).wait()
            wait(d3_hbm, CPT - 1, db, dsem_b)
            pltpu.sync_copy(rows_b, acc.at[db], add=True)

        plsc.subcore_barrier()
        pltpu.sync_copy(acc.at[pl.ds(base, RPT)],
                        out_hbm.at[cid, pl.ds(base, RPT)])

    return sc_scatter


def kernel(x, edge_index, batch, W_in, b_in, eps, W1, b1, g1, be1,
           W2, b2, g2, be2, Wh1, bh1, Wh2, bh2):
    src = edge_index[0]
    dst = edge_index[1]
    s3 = jnp.concatenate(
        [src, jnp.zeros((E_PAD - E,), jnp.int32)]).reshape(NW, CPT, C)
    pad_dst = N + jnp.arange(E_PAD - E, dtype=jnp.int32) % (N_ACC - N)
    d3 = jnp.concatenate([dst, pad_dst]).reshape(NW, CPT, C)
    b3 = batch.reshape(NB, 1, BLK)

    sc_scatter = _get_sc_scatter()

    h = _k0_call(x, W_in, b_in.reshape(1, D))
    for l in range(L):
        agg = sc_scatter(h, s3, d3)
        scale = (1.0 + eps[l]).reshape(1, 1)
        y1, s1 = _ka_call(scale, h, agg, W1[l], b1[l].reshape(1, D))
        y2, s2 = _kb_call(y1, s1, g1[l].reshape(1, D), be1[l].reshape(1, D),
                          W2[l], b2[l].reshape(1, D))
        h = _kc_call(y2, s2, g2[l].reshape(1, D), be2[l].reshape(1, D))

    pooled, cnt = _kpool_call(h, b3)
    out = _khead_call(pooled, cnt, Wh1, bh1.reshape(1, D // 2),
                      Wh2, bh2.reshape(1, 1))
    return out


# speedup vs baseline: 1.2525x; 1.2525x over previous
import functools

import jax
import jax.numpy as jnp
from jax import lax
from jax.experimental import pallas as pl
from jax.experimental.pallas import tpu as pltpu
from jax.experimental.pallas import tpu_sc as plsc

N = 10000
D = 128
L = 5
G = 256
E = 320000

BLK = 1000
NB = N // BLK

NW = 32
C = 128
CPT = 80
E_PAD = NW * C * CPT
N_ACC = 10240
RPT = N_ACC // 16


def _k0_body(x_ref, w_ref, b_ref, h_ref):
    h_ref[...] = jnp.maximum(
        jnp.dot(x_ref[...], w_ref[...], preferred_element_type=jnp.float32,
                precision=lax.Precision.HIGHEST)
        + b_ref[...], 0.0)


def _stats_update(i, s_ref, y):
    @pl.when(i == 0)
    def _():
        c0 = jnp.sum(y, axis=0, keepdims=True) * (1.0 / BLK)
        s_ref[...] = jnp.concatenate(
            [jnp.zeros((2, D), jnp.float32), c0,
             jnp.zeros((5, D), jnp.float32)], axis=0)

    c = s_ref[2:3, :]
    yc = y - c
    s_ref[...] += jnp.concatenate(
        [jnp.sum(yc, axis=0, keepdims=True),
         jnp.sum(yc * yc, axis=0, keepdims=True),
         jnp.zeros((6, D), jnp.float32)], axis=0)


def _ka_body(scale_ref, h_ref, a0_ref, a1_ref, w_ref, b_ref, y_ref, s_ref):
    i = pl.program_id(0)
    z = scale_ref[0, 0] * h_ref[...] + a0_ref[...] + a1_ref[...]
    y = jnp.dot(z, w_ref[...], preferred_element_type=jnp.float32,
                precision=lax.Precision.HIGHEST) + b_ref[...]
    y_ref[...] = y
    _stats_update(i, s_ref, y)


def _bn_relu(y, s_ref, g_ref, be_ref):
    s = s_ref[...]
    dm = s[0:1] * (1.0 / N)
    m = s[2:3] + dm
    v = s[1:2] * (1.0 / N) - dm * dm
    inv = 1.0 / jnp.sqrt(v + 1e-5)
    return jnp.maximum(g_ref[...] * (y - m) * inv + be_ref[...], 0.0)


def _kb_body(y1_ref, s1_ref, g_ref, be_ref, w_ref, b_ref, y2_ref, s2_ref):
    i = pl.program_id(0)
    t = _bn_relu(y1_ref[...], s1_ref, g_ref, be_ref)
    y2 = jnp.dot(t, w_ref[...], preferred_element_type=jnp.float32,
                precision=lax.Precision.HIGHEST) + b_ref[...]
    y2_ref[...] = y2
    _stats_update(i, s2_ref, y2)


def _kc_body(y2_ref, s2_ref, g_ref, be_ref, h_ref):
    h_ref[...] = _bn_relu(y2_ref[...], s2_ref, g_ref, be_ref)


def _kpool_body(h_ref, b3_ref, p_ref, c_ref):
    i = pl.program_id(0)
    bv = b3_ref[0]
    iota = lax.broadcasted_iota(jnp.int32, (G, BLK), 0)
    oh = (iota == bv).astype(jnp.float32)

    @pl.when(i == 0)
    def _():
        p_ref[...] = jnp.zeros_like(p_ref)
        c_ref[...] = jnp.zeros_like(c_ref)

    p_ref[...] += jnp.dot(oh, h_ref[...], preferred_element_type=jnp.float32,
                precision=lax.Precision.HIGHEST)
    c_ref[...] += jnp.concatenate(
        [jnp.sum(oh, axis=1, keepdims=True), jnp.zeros((G, 7), jnp.float32)],
        axis=1)


def _khead_body(p_ref, c_ref, w1_ref, b1_ref, w2_ref, b2_ref, o_ref):
    cnt = jnp.maximum(c_ref[...][:, 0:1], 1.0)
    mean = p_ref[...] / cnt
    t = jnp.maximum(
        jnp.dot(mean, w1_ref[...], preferred_element_type=jnp.float32,
                precision=lax.Precision.HIGHEST)
        + b1_ref[...], 0.0)
    o_ref[...] = (
        jnp.dot(t, w2_ref[...], preferred_element_type=jnp.float32,
                precision=lax.Precision.HIGHEST)
        + b2_ref[...])


def _full(shape):
    return pl.BlockSpec(shape, lambda i: (0,) * len(shape))


_ROWBLK = pl.BlockSpec((BLK, D), lambda i: (i, 0))


def _k0_call(x, w, b):
    return pl.pallas_call(
        _k0_body, grid=(NB,),
        in_specs=[_ROWBLK, _full((D, D)), _full((1, D))],
        out_specs=_ROWBLK,
        out_shape=jax.ShapeDtypeStruct((N, D), jnp.float32))(x, w, b)


def _ka_call(scale, h, a0, a1, w, b):
    return pl.pallas_call(
        _ka_body, grid=(NB,),
        in_specs=[
            _full((1, 1)),
            _ROWBLK,
            _ROWBLK,
            _ROWBLK,
            _full((D, D)),
            _full((1, D)),
        ],
        out_specs=[_ROWBLK, pl.BlockSpec((8, D), lambda i: (0, 0))],
        out_shape=[
            jax.ShapeDtypeStruct((N, D), jnp.float32),
            jax.ShapeDtypeStruct((8, D), jnp.float32),
        ])(scale, h, a0, a1, w, b)


def _kb_call(y1, s1, g, be, w, b):
    return pl.pallas_call(
        _kb_body, grid=(NB,),
        in_specs=[
            _ROWBLK, _full((8, D)), _full((1, D)), _full((1, D)),
            _full((D, D)), _full((1, D)),
        ],
        out_specs=[_ROWBLK, pl.BlockSpec((8, D), lambda i: (0, 0))],
        out_shape=[
            jax.ShapeDtypeStruct((N, D), jnp.float32),
            jax.ShapeDtypeStruct((8, D), jnp.float32),
        ])(y1, s1, g, be, w, b)


def _kc_call(y2, s2, g, be):
    return pl.pallas_call(
        _kc_body, grid=(NB,),
        in_specs=[_ROWBLK, _full((8, D)), _full((1, D)), _full((1, D))],
        out_specs=_ROWBLK,
        out_shape=jax.ShapeDtypeStruct((N, D), jnp.float32))(y2, s2, g, be)


def _kpool_call(h, b3):
    return pl.pallas_call(
        _kpool_body, grid=(NB,),
        in_specs=[_ROWBLK, pl.BlockSpec((1, 1, BLK), lambda i: (i, 0, 0))],
        out_specs=[
            pl.BlockSpec((G, D), lambda i: (0, 0)),
            pl.BlockSpec((G, 8), lambda i: (0, 0)),
        ],
        out_shape=[
            jax.ShapeDtypeStruct((G, D), jnp.float32),
            jax.ShapeDtypeStruct((G, 8), jnp.float32),
        ])(h, b3)


def _khead_call(p, c, w1, b1, w2, b2):
    return pl.pallas_call(
        _khead_body,
        out_shape=jax.ShapeDtypeStruct((G, 1), jnp.float32))(
            p, c, w1, b1, w2, b2)


@functools.cache
def _get_sc_scatter():
    mesh = plsc.VectorSubcoreMesh(core_axis_name="c", subcore_axis_name="s")

    @functools.partial(
        pl.kernel, mesh=mesh,
        out_type=[jax.ShapeDtypeStruct((N_ACC, D), jnp.float32),
                  jax.ShapeDtypeStruct((N_ACC, D), jnp.float32)],
        scratch_types=[
            pltpu.VMEM((C,), jnp.int32),
            pltpu.VMEM((C,), jnp.int32),
            pltpu.VMEM((C,), jnp.int32),
            pltpu.VMEM((C,), jnp.int32),
            pltpu.VMEM((C, D), jnp.float32),
            pltpu.VMEM((C, D), jnp.float32),
            pltpu.VMEM((16, D), jnp.float32),
            pltpu.VMEM_SHARED((N_ACC, D), jnp.float32),
            pltpu.SemaphoreType.DMA,
            pltpu.SemaphoreType.DMA,
            pltpu.SemaphoreType.DMA,
            pltpu.SemaphoreType.DMA,
            pltpu.SemaphoreType.DMA,
            pltpu.SemaphoreType.DMA,
        ])
    def sc_scatter(h_hbm, s3_hbm, d3_hbm, out0_hbm, out1_hbm,
                   sa, da, sb, db, rows_a, rows_b, zrows, acc,
                   ssem_a, dsem_a, ssem_b, dsem_b, gsem_a, gsem_b):
        cid = lax.axis_index("c")
        sid = lax.axis_index("s")
        wid = cid * 16 + sid

        zero16 = jnp.zeros((16,), jnp.float32)
        for r in range(16):
            for c8 in range(8):
                zrows[r, pl.ds(c8 * 16, 16)] = zero16
        base = sid * RPT
        for j in range(RPT // 16):
            pltpu.sync_copy(zrows, acc.at[pl.ds(base + j * 16, 16)])
        plsc.subcore_barrier()

        def fetch(ref3, i, buf, sem):
            pltpu.async_copy(ref3.at[wid, i], buf, sem)

        def wait(ref3, i, buf, sem):
            pltpu.make_async_copy(ref3.at[wid, i], buf, sem).wait()

        def _edge_work():
            fetch(s3_hbm, 0, sa, ssem_a)
            fetch(d3_hbm, 0, da, dsem_a)
            fetch(s3_hbm, 1, sb, ssem_b)
            fetch(d3_hbm, 1, db, dsem_b)
            wait(s3_hbm, 0, sa, ssem_a)
            pltpu.async_copy(h_hbm.at[sa], rows_a, gsem_a)
            wait(s3_hbm, 1, sb, ssem_b)
            pltpu.async_copy(h_hbm.at[sb], rows_b, gsem_b)

            def slot_step(i, s_buf, d_buf, rows, ssem, dsem, gsem):
                pltpu.make_async_copy(h_hbm.at[s_buf], rows, gsem).wait()
                fetch(s3_hbm, i + 2, s_buf, ssem)
                wait(d3_hbm, i, d_buf, dsem)
                pltpu.sync_copy(rows, acc.at[d_buf], add=True)
                fetch(d3_hbm, i + 2, d_buf, dsem)
                wait(s3_hbm, i + 2, s_buf, ssem)
                pltpu.async_copy(h_hbm.at[s_buf], rows, gsem)

            def body(j, carry):
                i0 = 2 * j
                slot_step(i0, sa, da, rows_a, ssem_a, dsem_a, gsem_a)
                slot_step(i0 + 1, sb, db, rows_b, ssem_b, dsem_b, gsem_b)
                return carry

            lax.fori_loop(0, CPT // 2 - 1, body, 0)
            pltpu.make_async_copy(h_hbm.at[sa], rows_a, gsem_a).wait()
            wait(d3_hbm, CPT - 2, da, dsem_a)
            pltpu.sync_copy(rows_a, acc.at[da], add=True)
            pltpu.make_async_copy(h_hbm.at[sb], rows_b, gsem_b).wait()
            wait(d3_hbm, CPT - 1, db, dsem_b)
            pltpu.sync_copy(rows_b, acc.at[db], add=True)

        _edge_work()
        plsc.subcore_barrier()

        @pl.when(cid == 0)
        def _():
            pltpu.sync_copy(acc.at[pl.ds(base, RPT)],
                            out0_hbm.at[pl.ds(base, RPT)])

        @pl.when(cid == 1)
        def _():
            pltpu.sync_copy(acc.at[pl.ds(base, RPT)],
                            out1_hbm.at[pl.ds(base, RPT)])

    return sc_scatter


def kernel(x, edge_index, batch, W_in, b_in, eps, W1, b1, g1, be1,
           W2, b2, g2, be2, Wh1, bh1, Wh2, bh2):
    src = edge_index[0]
    dst = edge_index[1]
    s3 = jnp.concatenate(
        [src, jnp.zeros((E_PAD - E,), jnp.int32)]).reshape(NW, CPT, C)
    pad_dst = N + jnp.arange(E_PAD - E, dtype=jnp.int32) % (N_ACC - N)
    d3 = jnp.concatenate([dst, pad_dst]).reshape(NW, CPT, C)
    b3 = batch.reshape(NB, 1, BLK)

    sc_scatter = _get_sc_scatter()

    h = _k0_call(x, W_in, b_in.reshape(1, D))
    for l in range(L):
        a0, a1 = sc_scatter(h, s3, d3)
        scale = (1.0 + eps[l]).reshape(1, 1)
        y1, s1 = _ka_call(scale, h, a0, a1, W1[l], b1[l].reshape(1, D))
        y2, s2 = _kb_call(y1, s1, g1[l].reshape(1, D), be1[l].reshape(1, D),
                          W2[l], b2[l].reshape(1, D))
        h = _kc_call(y2, s2, g2[l].reshape(1, D), be2[l].reshape(1, D))

    pooled, cnt = _kpool_call(h, b3)
    out = _khead_call(pooled, cnt, Wh1, bh1.reshape(1, D // 2),
                      Wh2, bh2.reshape(1, 1))
    return out
